# baseline (speedup 1.0000x reference)
import functools

import jax
import jax.numpy as jnp
from jax.experimental import pallas as pl
from jax.experimental.pallas import tpu as pltpu


def _scse_ring_kernel(x_hbm, w1_ref, b1_ref, w2_ref, b2_ref, ws_ref, bs_ref,
                      o_hbm, ibuf, obuf, in_sem, out_sem,
                      *, n_planes, kbuf, inv_hw):
    def start_in(n, slot):
        pltpu.make_async_copy(x_hbm.at[pl.ds(n, 1)], ibuf.at[pl.ds(slot, 1)],
                              in_sem.at[slot]).start()

    def wait_in(slot):
        pltpu.make_async_copy(x_hbm.at[pl.ds(0, 1)], ibuf.at[pl.ds(slot, 1)],
                              in_sem.at[slot]).wait()

    def start_out(n, slot):
        pltpu.make_async_copy(obuf.at[pl.ds(slot, 1)], o_hbm.at[pl.ds(n, 1)],
                              out_sem.at[slot]).start()

    def wait_out(slot):
        pltpu.make_async_copy(obuf.at[pl.ds(slot, 1)], o_hbm.at[pl.ds(0, 1)],
                              out_sem.at[slot]).wait()

    for k in range(kbuf):
        start_in(k, k)

    def body(n, _):
        slot = jax.lax.rem(n, kbuf)
        wait_in(slot)

        @pl.when(n >= kbuf)
        def _():
            wait_out(slot)

        x = ibuf[pl.ds(slot, 1)][0]

        mean = jnp.sum(x, axis=1, keepdims=True) * inv_hw
        z = jnp.sum(mean * w1_ref[...], axis=0, keepdims=True)
        z = jnp.maximum(z + b1_ref[...], 0.0)
        s = jnp.sum(w2_ref[...] * z, axis=1, keepdims=True)
        s = jax.nn.sigmoid(s + b2_ref[...])

        q = jax.lax.dot_general(ws_ref[...], x, (((0,), (0,)), ((), ())),
                                preferred_element_type=jnp.float32)
        q = jax.nn.sigmoid(q + bs_ref[0])

        obuf[pl.ds(slot, 1)] = (x * (s + q))[None]
        start_out(n, slot)

        @pl.when(n + kbuf < n_planes)
        def _():
            start_in(n + kbuf, slot)

        return 0

    jax.lax.fori_loop(0, n_planes, body, 0, unroll=False)

    for p in range(max(n_planes - kbuf, 0), n_planes):
        wait_out(p % kbuf)


def kernel(x_nchw, w1, b1, w2, b2, ws, bs):
    N, C, H, W = x_nchw.shape
    HW = H * W
    dtype = x_nchw.dtype
    x = x_nchw.reshape(N, C, HW)

    HWp = ((HW + 127) // 128) * 128
    if HWp != HW:
        x = jnp.pad(x, ((0, 0), (0, 0), (0, HWp - HW)))

    kbuf = min(3, N)

    body = functools.partial(_scse_ring_kernel, n_planes=N, kbuf=kbuf,
                             inv_hw=1.0 / float(HW))
    out = pl.pallas_call(
        body,
        out_shape=jax.ShapeDtypeStruct((N, C, HWp), dtype),
        in_specs=[
            pl.BlockSpec(memory_space=pltpu.MemorySpace.HBM),
            pl.BlockSpec(memory_space=pltpu.MemorySpace.VMEM),
            pl.BlockSpec(memory_space=pltpu.MemorySpace.VMEM),
            pl.BlockSpec(memory_space=pltpu.MemorySpace.VMEM),
            pl.BlockSpec(memory_space=pltpu.MemorySpace.VMEM),
            pl.BlockSpec(memory_space=pltpu.MemorySpace.VMEM),
            pl.BlockSpec(memory_space=pltpu.MemorySpace.SMEM),
        ],
        out_specs=pl.BlockSpec(memory_space=pltpu.MemorySpace.HBM),
        scratch_shapes=[
            pltpu.VMEM((kbuf, C, HWp), jnp.float32),
            pltpu.VMEM((kbuf, C, HWp), jnp.float32),
            pltpu.SemaphoreType.DMA((kbuf,)),
            pltpu.SemaphoreType.DMA((kbuf,)),
        ],
        compiler_params=pltpu.CompilerParams(
            vmem_limit_bytes=60 * 1024 * 1024,
        ),
        cost_estimate=pl.CostEstimate(
            flops=6 * N * C * HWp,
            transcendentals=N * (HWp + C),
            bytes_accessed=2 * N * C * HWp * dtype.itemsize,
        ),
    )(x, w1, b1, w2, b2, ws, bs)

    if HWp != HW:
        out = out[:, :, :HW]
    return out.reshape(N, C, H, W)
